# baseline (speedup 1.0000x reference)
import functools

import jax
import jax.numpy as jnp
from jax import lax
from jax.experimental import pallas as pl
from jax.experimental.pallas import tpu as pltpu
from jax.experimental.pallas import tpu_sc as plsc

_NUM_CORES = 2
_NUM_SUBCORES = 16
_NUM_WORKERS = _NUM_CORES * _NUM_SUBCORES
_LANES = 16
_CHUNK = 128


def _sc_body(batch, vocab, n_ids, scores_hbm, ids_hbm, out_hbm,
             ids_v, fill_v, vals_v, loc_v, idx2_v, idxe_v, isem, fsem, gsem):
    rows = vocab // _NUM_WORKERS
    fill_rows = fill_v.shape[0]
    n_fill = (rows - 7) // fill_rows
    interior = n_fill * fill_rows
    n_vecs = n_ids // _LANES
    loc_vecs = loc_v.shape[0] // _LANES

    cid = lax.axis_index("c")
    sid = lax.axis_index("s")
    wid = sid * _NUM_CORES + cid
    lo = wid * rows
    hi = lo + rows
    a0 = (lo + 7) // 8 * 8
    head = a0 - lo

    ids_cp = pltpu.async_copy(ids_hbm, ids_v, isem)

    c16 = jnp.arange(_LANES, dtype=jnp.int32)

    neg_inf = jnp.full((_LANES,), -jnp.inf, dtype=jnp.float32)

    def fill_init(i, carry):
        fill_v[i // 8, pl.ds((i % 8) * _LANES, _LANES)] = neg_inf
        return carry

    lax.fori_loop(0, fill_rows * batch // _LANES, fill_init, 0)

    def edge_idx(c, carry):
        j = c * _LANES + c16
        row = jnp.where(j < head, lo + j, a0 + interior + (j - head))
        idxe_v[0, pl.ds(c * _LANES, _LANES)] = jnp.minimum(row, hi - 1)
        return carry

    lax.fori_loop(0, _CHUNK // _LANES, edge_idx, 0)

    fdescs = [
        pltpu.async_copy(
            fill_v, out_hbm.at[pl.ds(a0 + k * fill_rows, fill_rows)], fsem)
        for k in range(n_fill)
    ]
    fdescs.append(
        pltpu.async_copy(fill_v.at[pl.ds(0, _CHUNK)],
                         out_hbm.at[idxe_v.at[0]], fsem))

    ids_cp.wait()

    def compact(i, k):
        v = ids_v[pl.ds(i * _LANES, _LANES)]
        m = (v >= lo) & (v < hi)
        plsc.store_compressed(loc_v.at[pl.ds(k, _LANES)], v, mask=m)
        return k + jnp.sum(m.astype(jnp.int32))

    n_local = lax.fori_loop(0, n_vecs, compact, 0)

    for d in fdescs:
        d.wait()

    @pl.when(n_local > 0)
    def _():
        first = loc_v[pl.ds(0, _LANES)]
        pad_id = jnp.min(jnp.where(c16 < n_local, first, jnp.int32(2**30)))

        def pad(c, carry):
            cur = loc_v[pl.ds(c * _LANES, _LANES)]
            keep = (c * _LANES + c16) < n_local
            loc_v[pl.ds(c * _LANES, _LANES)] = jnp.where(
                keep, cur, jnp.zeros_like(cur) + pad_id)
            return carry

        lax.fori_loop(0, loc_vecs, pad, 0)

        def mirror(c, carry):
            idx2_v[c // 8, pl.ds((c % 8) * _LANES, _LANES)] = (
                loc_v[pl.ds(c * _LANES, _LANES)])
            return carry

        lax.fori_loop(0, loc_vecs, mirror, 0)

        n_chunks = (n_local + _CHUNK - 1) // _CHUNK

        def move(c, carry):
            pltpu.async_copy(scores_hbm.at[idx2_v.at[c]], vals_v, gsem).wait()
            pltpu.async_copy(vals_v, out_hbm.at[idx2_v.at[c]], gsem).wait()
            return carry

        lax.fori_loop(0, n_chunks, move, 0)


def kernel(input_ids, scores, allowed_token_ids):
    del input_ids
    batch, vocab = scores.shape
    n_ids = allowed_token_ids.shape[0]
    ids = allowed_token_ids.astype(jnp.int32)
    scores_t = scores.T

    rows = vocab // _NUM_WORKERS
    fill_rows = 32
    assert rows - (rows - 7) // fill_rows * fill_rows <= _CHUNK
    loc_cap = n_ids + _CHUNK
    mesh = plsc.VectorSubcoreMesh(core_axis_name="c", subcore_axis_name="s")
    out_t = pl.kernel(
        functools.partial(_sc_body, batch, vocab, n_ids),
        out_type=jax.ShapeDtypeStruct((vocab, batch), jnp.float32),
        mesh=mesh,
        scratch_types=[
            pltpu.VMEM((n_ids,), jnp.int32),
            pltpu.VMEM((fill_rows, batch), jnp.float32),
            pltpu.VMEM((_CHUNK, batch), jnp.float32),
            pltpu.VMEM((loc_cap,), jnp.int32),
            pltpu.VMEM((loc_cap // _CHUNK, _CHUNK), jnp.int32),
            pltpu.VMEM((1, _CHUNK), jnp.int32),
            pltpu.SemaphoreType.DMA,
            pltpu.SemaphoreType.DMA,
            pltpu.SemaphoreType.DMA,
        ],
        compiler_params=pltpu.CompilerParams(needs_layout_passes=False),
        name="sc_sparse_logits_mask",
    )(scores_t, ids)
    return out_t.T


# speedup vs baseline: 1.1762x; 1.1762x over previous
import functools

import jax
import jax.numpy as jnp
from jax import lax
from jax.experimental import pallas as pl
from jax.experimental.pallas import tpu as pltpu
from jax.experimental.pallas import tpu_sc as plsc

_NUM_CORES = 2
_NUM_SUBCORES = 16
_NUM_WORKERS = _NUM_CORES * _NUM_SUBCORES
_LANES = 16
_CHUNK = 128
_FILL_BLK = 6400


def _fill_body(out_ref):
    out_ref[...] = jnp.full(out_ref.shape, -jnp.inf, dtype=jnp.float32)


def _sc_scatter(vocab, scores_hbm, ids_hbm, out_hbm,
                ids_v, vals_v, loc_v, idx2_v, isem, gsem):
    n_ids = ids_v.shape[0]
    rows = vocab // _NUM_WORKERS
    n_vecs = n_ids // _LANES
    loc_vecs = loc_v.shape[0] // _LANES

    cid = lax.axis_index("c")
    sid = lax.axis_index("s")
    wid = sid * _NUM_CORES + cid
    lo = wid * rows
    hi = lo + rows

    pltpu.async_copy(ids_hbm, ids_v, isem).wait()

    c16 = jnp.arange(_LANES, dtype=jnp.int32)

    def compact(i, k):
        v = ids_v[pl.ds(i * _LANES, _LANES)]
        m = (v >= lo) & (v < hi)
        plsc.store_compressed(loc_v.at[pl.ds(k, _LANES)], v, mask=m)
        return k + jnp.sum(m.astype(jnp.int32))

    n_local = lax.fori_loop(0, n_vecs, compact, 0)

    @pl.when(n_local > 0)
    def _():
        first = loc_v[pl.ds(0, _LANES)]
        pad_id = jnp.min(jnp.where(c16 < n_local, first, jnp.int32(2**30)))

        def pad(c, carry):
            cur = loc_v[pl.ds(c * _LANES, _LANES)]
            keep = (c * _LANES + c16) < n_local
            loc_v[pl.ds(c * _LANES, _LANES)] = jnp.where(
                keep, cur, jnp.zeros_like(cur) + pad_id)
            return carry

        lax.fori_loop(0, loc_vecs, pad, 0)

        def mirror(c, carry):
            idx2_v[c // 8, pl.ds((c % 8) * _LANES, _LANES)] = (
                loc_v[pl.ds(c * _LANES, _LANES)])
            return carry

        lax.fori_loop(0, loc_vecs, mirror, 0)

        n_chunks = (n_local + _CHUNK - 1) // _CHUNK

        def move(c, carry):
            pltpu.async_copy(scores_hbm.at[idx2_v.at[c]], vals_v, gsem).wait()
            pltpu.async_copy(vals_v, out_hbm.at[idx2_v.at[c]], gsem).wait()
            return carry

        lax.fori_loop(0, n_chunks, move, 0)


def kernel(input_ids, scores, allowed_token_ids):
    del input_ids
    batch, vocab = scores.shape
    n_ids = allowed_token_ids.shape[0]
    ids = allowed_token_ids.astype(jnp.int32)
    scores_t = scores.T

    filled_t = pl.pallas_call(
        _fill_body,
        grid=(-(-vocab // _FILL_BLK),),
        out_specs=pl.BlockSpec((_FILL_BLK, batch), lambda i: (i, 0)),
        out_shape=jax.ShapeDtypeStruct((vocab, batch), jnp.float32),
        name="tc_neg_inf_fill",
    )()

    loc_cap = n_ids + _CHUNK
    mesh = plsc.VectorSubcoreMesh(core_axis_name="c", subcore_axis_name="s")

    def run(refs):
        scores_ref, ids_ref, out_ref = refs

        @pl.core_map(
            mesh,
            compiler_params=pltpu.CompilerParams(needs_layout_passes=False),
            scratch_shapes=[
                pltpu.VMEM((n_ids,), jnp.int32),
                pltpu.VMEM((_CHUNK, batch), jnp.float32),
                pltpu.VMEM((loc_cap,), jnp.int32),
                pltpu.VMEM((loc_cap // _CHUNK, _CHUNK), jnp.int32),
                pltpu.SemaphoreType.DMA,
                pltpu.SemaphoreType.DMA,
            ],
            name="sc_scatter_allowed_rows",
        )
        def _(*scratch):
            _sc_scatter(vocab, scores_ref, ids_ref, out_ref, *scratch)

    _, _, out_t = pl.run_state(run)((scores_t, ids, filled_t))
    return out_t.T


# speedup vs baseline: 1.5858x; 1.3482x over previous
import jax
import jax.numpy as jnp
from jax import lax
from jax.experimental import pallas as pl
from jax.experimental.pallas import tpu as pltpu
from jax.experimental.pallas import tpu_sc as plsc

_NUM_CORES = 2
_NUM_SUBCORES = 16
_NUM_WORKERS = _NUM_CORES * _NUM_SUBCORES
_FILL_BLK = 6400


def _worker_id():
    return lax.axis_index("s") * _NUM_CORES + lax.axis_index("c")


def _sc_gather_body(per_w, scores_hbm, ids_hbm, staged_hbm,
                    idx_v, vals_v, isem, gsem):
    base = _worker_id() * per_w
    pltpu.async_copy(ids_hbm.at[pl.ds(base, per_w)], idx_v.at[0], isem).wait()
    pltpu.async_copy(scores_hbm.at[idx_v.at[0]], vals_v, gsem).wait()
    pltpu.async_copy(vals_v, staged_hbm.at[pl.ds(base, per_w)], gsem).wait()


def _sc_scatter_body(per_w, ids_hbm, staged_hbm, out_hbm,
                     idx_v, vals_v, isem, gsem):
    base = _worker_id() * per_w
    ids_cp = pltpu.async_copy(ids_hbm.at[pl.ds(base, per_w)], idx_v.at[0], isem)
    vals_cp = pltpu.async_copy(
        staged_hbm.at[pl.ds(base, per_w)], vals_v, gsem)
    ids_cp.wait()
    vals_cp.wait()
    pltpu.async_copy(vals_v, out_hbm.at[idx_v.at[0]], gsem).wait()


def _fill_body(out_ref):
    out_ref[...] = jnp.full(out_ref.shape, -jnp.inf, dtype=jnp.float32)


def kernel(input_ids, scores, allowed_token_ids):
    del input_ids
    batch, vocab = scores.shape
    n_ids = allowed_token_ids.shape[0]
    per_w = n_ids // _NUM_WORKERS
    ids = allowed_token_ids.astype(jnp.int32)
    scores_t = scores.T

    mesh = plsc.VectorSubcoreMesh(core_axis_name="c", subcore_axis_name="s")
    sc_params = pltpu.CompilerParams(needs_layout_passes=False)

    def gather_body(*args):
        _sc_gather_body(per_w, *args)

    staged = pl.kernel(
        gather_body,
        out_type=jax.ShapeDtypeStruct((n_ids, batch), jnp.float32),
        mesh=mesh,
        scratch_types=[
            pltpu.VMEM((1, per_w), jnp.int32),
            pltpu.VMEM((per_w, batch), jnp.float32),
            pltpu.SemaphoreType.DMA,
            pltpu.SemaphoreType.DMA,
        ],
        compiler_params=sc_params,
        name="sc_gather_allowed_rows",
    )(scores_t, ids)

    filled_t = pl.pallas_call(
        _fill_body,
        grid=(-(-vocab // _FILL_BLK),),
        out_specs=pl.BlockSpec((_FILL_BLK, batch), lambda i: (i, 0)),
        out_shape=jax.ShapeDtypeStruct((vocab, batch), jnp.float32),
        name="tc_neg_inf_fill",
    )()

    def run(refs):
        ids_ref, staged_ref, out_ref = refs

        @pl.core_map(
            mesh,
            compiler_params=sc_params,
            scratch_shapes=[
                pltpu.VMEM((1, per_w), jnp.int32),
                pltpu.VMEM((per_w, batch), jnp.float32),
                pltpu.SemaphoreType.DMA,
                pltpu.SemaphoreType.DMA,
            ],
            name="sc_scatter_allowed_rows",
        )
        def _(*scratch):
            _sc_scatter_body(per_w, ids_ref, staged_ref, out_ref, *scratch)

    _, _, out_t = pl.run_state(run)((ids, staged, filled_t))
    return out_t.T
